# Initial kernel scaffold; baseline (speedup 1.0000x reference)
import jax
import jax.numpy as jnp
from jax.experimental import pallas as pl


def kernel(logits, tgt_in_idx):
    raise NotImplementedError("write your pallas kernel here")


# speedup vs baseline: 7.0981x; 7.0981x over previous
import functools

import jax
import jax.numpy as jnp
from jax.experimental import pallas as pl
from jax.experimental.pallas import tpu as pltpu

B, S, C = 128, 32, 8192
NEG_INF = float("-inf")


def _tc_body(idx_ref, x_ref, o_ref):
    x = x_ref[0]
    idx = idx_ref[0]
    cand = jax.lax.broadcasted_iota(jnp.int32, (S, C), 1)
    eq = (cand == idx).astype(jnp.float32)
    row = jax.lax.broadcasted_iota(jnp.int32, (S, S), 0)
    col = jax.lax.broadcasted_iota(jnp.int32, (S, S), 1)
    tril = (row >= col).astype(jnp.float32)
    counts = jnp.dot(tril, eq, preferred_element_type=jnp.float32)
    mask = (counts > 0.0) | (cand < 2)
    masked = jnp.where(mask, NEG_INF, x)
    m = jnp.max(masked, axis=1, keepdims=True)
    s = jnp.sum(jnp.exp(masked - m), axis=1, keepdims=True)
    o_ref[0] = masked - (m + jnp.log(s))


@jax.jit
def kernel(logits, tgt_in_idx):
    idx3 = tgt_in_idx[:, :, None]
    return pl.pallas_call(
        _tc_body,
        grid=(B,),
        in_specs=[
            pl.BlockSpec((1, S, 1), lambda b: (b, 0, 0)),
            pl.BlockSpec((1, S, C), lambda b: (b, 0, 0)),
        ],
        out_specs=pl.BlockSpec((1, S, C), lambda b: (b, 0, 0)),
        out_shape=jax.ShapeDtypeStruct((B, S, C), jnp.float32),
        compiler_params=pltpu.CompilerParams(
            dimension_semantics=("arbitrary",),
        ),
    )(idx3, logits)
